# baseline (speedup 1.0000x reference)
import jax
import jax.numpy as jnp
from jax import lax
from jax.experimental import pallas as pl
from jax.experimental.pallas import tpu as pltpu
from jax.experimental.pallas import tpu_sc as plsc

NC = 2
NS = 16
NW = NC * NS
L = 16

B = 16384
D = 64
BPW = B // NW
GCHUNK = 128
NCHUNK = BPW // GCHUNK
RCHUNK = 16
NRED = BPW // RCHUNK


def _body(user_hbm, item_hbm, ufac_hbm, ifac_hbm,
          pred_hbm, qu_hbm, pi_hbm,
          uidx_v, iidx_v, urows_v, irows_v, pred_v, qu_v, pi_v, sem):
    wid = lax.axis_index("s") * NC + lax.axis_index("c")
    base = wid * BPW

    pltpu.sync_copy(user_hbm.at[pl.ds(base, BPW)], uidx_v)
    pltpu.sync_copy(item_hbm.at[pl.ds(base, BPW)], iidx_v)

    copies = []
    for k in range(NCHUNK):
        s = pl.ds(k * GCHUNK, GCHUNK)
        copies.append(pltpu.async_copy(ufac_hbm.at[uidx_v.at[s]],
                                       urows_v.at[s], sem))
        copies.append(pltpu.async_copy(ifac_hbm.at[iidx_v.at[s]],
                                       irows_v.at[s], sem))
    for c in copies:
        c.wait()

    lane = lax.iota(jnp.int32, L)

    def chunk_body(c, carry):
        predv = jnp.zeros((L,), jnp.float32)
        quv = jnp.zeros((L,), jnp.float32)
        piv = jnp.zeros((L,), jnp.float32)
        for j in range(RCHUNK):
            r = c * RCHUNK + j
            p = jnp.zeros((L,), jnp.float32)
            q = jnp.zeros((L,), jnp.float32)
            t = jnp.zeros((L,), jnp.float32)
            for d in range(D // L):
                u = urows_v[r, pl.ds(d * L, L)]
                v = irows_v[r, pl.ds(d * L, L)]
                p = p + u * v
                q = q + u * u
                t = t + v * v
            predv = jnp.where(lane == j, jnp.sum(p), predv)
            quv = jnp.where(lane == j, jnp.sum(q), quv)
            piv = jnp.where(lane == j, jnp.sum(t), piv)
        pred_v[pl.ds(c * RCHUNK, RCHUNK)] = predv
        qu_v[pl.ds(c * RCHUNK, RCHUNK)] = quv
        pi_v[pl.ds(c * RCHUNK, RCHUNK)] = piv
        return carry

    lax.fori_loop(0, NRED, chunk_body, 0)

    pltpu.sync_copy(pred_v, pred_hbm.at[pl.ds(base, BPW)])
    pltpu.sync_copy(qu_v, qu_hbm.at[pl.ds(base, BPW)])
    pltpu.sync_copy(pi_v, pi_hbm.at[pl.ds(base, BPW)])


_sc_call = pl.kernel(
    _body,
    out_type=(
        jax.ShapeDtypeStruct((B,), jnp.float32),
        jax.ShapeDtypeStruct((B,), jnp.float32),
        jax.ShapeDtypeStruct((B,), jnp.float32),
    ),
    mesh=plsc.VectorSubcoreMesh(core_axis_name="c", subcore_axis_name="s",
                                num_cores=NC, num_subcores=NS),
    compiler_params=pltpu.CompilerParams(needs_layout_passes=False,
                                         use_tc_tiling_on_sc=False),
    scratch_types=[
        pltpu.VMEM((BPW,), jnp.int32),
        pltpu.VMEM((BPW,), jnp.int32),
        pltpu.VMEM((BPW, D), jnp.float32),
        pltpu.VMEM((BPW, D), jnp.float32),
        pltpu.VMEM((BPW,), jnp.float32),
        pltpu.VMEM((BPW,), jnp.float32),
        pltpu.VMEM((BPW,), jnp.float32),
        pltpu.SemaphoreType.DMA,
    ],
)


@jax.jit
def kernel(user, item, user_factors, item_factors):
    return _sc_call(user, item, user_factors, item_factors)
